# Initial kernel scaffold; baseline (speedup 1.0000x reference)
import jax
import jax.numpy as jnp
from jax.experimental import pallas as pl


def kernel(coords, features, params):
    raise NotImplementedError("write your pallas kernel here")


# speedup vs baseline: 2.2470x; 2.2470x over previous
import functools

import jax
import jax.numpy as jnp
import numpy as np
from jax import lax
from jax.experimental import pallas as pl
from jax.experimental.pallas import tpu as pltpu

IMG = 256
NB = 4
EPS = 1e-4
R_STRIP = 64
NPAD = 102400


def _bn_ab(bn, cnt):
    mean = bn[0] * (1.0 / cnt)
    var = bn[1] * (1.0 / cnt) - mean * mean
    a = bn[2] * lax.rsqrt(var + EPS)
    b = bn[3] - mean * a
    return a, b


def _stats_update(ostats_ref, b, acc1, acc2):
    @pl.when(b == 0)
    def _():
        ostats_ref[...] = jnp.zeros_like(ostats_ref)
    ostats_ref[0, :] = ostats_ref[0, :] + acc1
    ostats_ref[1, :] = ostats_ref[1, :] + acc2


def _dot(a, b):
    return lax.dot_general(a, b, (((1,), (0,)), ((), ())),
                           preferred_element_type=jnp.float32)


def _shift3(x):
    C, rows, W = x.shape
    z = jnp.zeros((C, rows, 1), jnp.float32)
    xl = jnp.concatenate([z, x[:, :, : W - 1]], axis=2)
    xr = jnp.concatenate([x[:, :, 1:], z], axis=2)
    return jnp.concatenate([xl, x, xr], axis=0)


def _row_mask(rows, i0, Hp):
    ri = lax.broadcasted_iota(jnp.int32, (1, rows, 1), 1) + i0
    return (ri >= 1) & (ri <= Hp - 2)


def _fused_conv(xs, W3, Cout, H, cnt, res_ident=None, res_nin=None):
    Wd = H
    Hp = H + 2
    n_in = len(xs)
    n_nin = len(res_nin) if res_nin else 0
    has_bn = [bn is not None for _, bn in xs]
    Cis = [int(a.shape[1]) for a, _ in xs]
    Cres = [int(a.shape[1]) for a, _ in (res_nin or [])]
    R = min(R_STRIP, H)

    def body(*refs):
        i = 0
        x_refs = refs[i:i + n_in]; i += n_in
        bn_refs = []
        for hb in has_bn:
            bn_refs.append(refs[i] if hb else None)
            i += 1 if hb else 0
        w_ref = refs[i]; i += 1
        ri_ref = None
        if res_ident is not None:
            ri_ref = refs[i]; i += 1
        rn_refs = refs[i:i + n_nin]; i += n_nin
        rw_refs = refs[i:i + n_nin]; i += n_nin
        out_ref, ostats_ref = refs[i], refs[i + 1]

        b = pl.program_id(0)
        abs_ = [None if bn_refs[k] is None else _bn_ab(bn_refs[k][...], cnt)
                for k in range(n_in)]
        out_ref[0, :, 0, :] = jnp.zeros((Cout, Wd), jnp.float32)
        out_ref[0, :, Hp - 1, :] = jnp.zeros((Cout, Wd), jnp.float32)
        acc1 = jnp.zeros((Cout,), jnp.float32)
        acc2 = jnp.zeros((Cout,), jnp.float32)
        for i0 in range(0, H, R):
            parts = []
            for k in range(n_in):
                s = x_refs[k][0, :, i0:i0 + R + 2, :]
                if abs_[k] is not None:
                    a, bb = abs_[k]
                    s = jnp.maximum(s * a[:, None, None] + bb[:, None, None], 0.0)
                    s = jnp.where(_row_mask(R + 2, i0, Hp), s, 0.0)
                parts.append(s)
            xr = parts[0] if n_in == 1 else jnp.concatenate(parts, axis=0)
            Ct = xr.shape[0]
            xcat = _shift3(xr)
            y = None
            for dy in range(3):
                wdy = w_ref[dy]
                rhs = xcat[:, dy:dy + R, :].reshape(3 * Ct, R * Wd)
                p = _dot(wdy, rhs)
                y = p if y is None else y + p
            y = y.reshape(Cout, R, Wd)
            if ri_ref is not None:
                y = y + ri_ref[0, :, 1 + i0:1 + i0 + R, :]
            for k in range(n_nin):
                xres = rn_refs[k][0, :, 1 + i0:1 + i0 + R, :]
                y = y + _dot(rw_refs[k][...],
                             xres.reshape(Cres[k], R * Wd)).reshape(Cout, R, Wd)
            acc1 = acc1 + y.sum((1, 2))
            acc2 = acc2 + (y * y).sum((1, 2))
            out_ref[0, :, 1 + i0:1 + i0 + R, :] = y
        _stats_update(ostats_ref, b, acc1, acc2)

    operands = []
    in_specs = []

    def full(shape):
        return pl.BlockSpec(shape, lambda b: (0,) * len(shape))

    for arr, bn in xs:
        operands.append(arr)
        in_specs.append(pl.BlockSpec((1,) + arr.shape[1:],
                                     lambda b: (b, 0, 0, 0)))
    for arr, bn in xs:
        if bn is not None:
            operands.append(bn)
            in_specs.append(full(bn.shape))
    operands.append(W3); in_specs.append(full(W3.shape))
    if res_ident is not None:
        operands.append(res_ident)
        in_specs.append(pl.BlockSpec((1, Cout, Hp, Wd), lambda b: (b, 0, 0, 0)))
    if res_nin:
        for arr, _ in res_nin:
            operands.append(arr)
            in_specs.append(pl.BlockSpec((1,) + arr.shape[1:],
                                         lambda b: (b, 0, 0, 0)))
        for _, w in res_nin:
            operands.append(w); in_specs.append(full(w.shape))

    out0 = jax.ShapeDtypeStruct((NB, Cout, Hp, Wd), jnp.float32)
    spec0 = pl.BlockSpec((1, Cout, Hp, Wd), lambda b: (b, 0, 0, 0))
    out_shapes = (out0, jax.ShapeDtypeStruct((2, Cout), jnp.float32))
    out_specs = (spec0, pl.BlockSpec((2, Cout), lambda b: (0, 0)))
    return pl.pallas_call(body, grid=(NB,), in_specs=in_specs,
                          out_specs=out_specs, out_shape=out_shapes)(*operands)


def _stem(grid_parts, W3, Cout):
    H = Wd = IMG
    Hp = H + 2
    R = R_STRIP

    def body(x_ref, w_ref, out_ref, ostats_ref):
        b = pl.program_id(0)
        x = x_ref[0, 0] + x_ref[1, 0]
        zrow = jnp.zeros((1, Wd), jnp.float32)
        xp = jnp.concatenate([zrow, x, zrow], axis=0)[None]
        xcat = _shift3(xp)
        out_ref[0, :, 0, :] = jnp.zeros((Cout, Wd), jnp.float32)
        out_ref[0, :, Hp - 1, :] = jnp.zeros((Cout, Wd), jnp.float32)
        acc1 = jnp.zeros((Cout,), jnp.float32)
        acc2 = jnp.zeros((Cout,), jnp.float32)
        for i0 in range(0, H, R):
            y = None
            for dy in range(3):
                rhs = xcat[:, i0 + dy:i0 + dy + R, :].reshape(3, R * Wd)
                p = _dot(w_ref[dy], rhs)
                y = p if y is None else y + p
            y = y.reshape(Cout, R, Wd)
            acc1 = acc1 + y.sum((1, 2))
            acc2 = acc2 + (y * y).sum((1, 2))
            out_ref[0, :, 1 + i0:1 + i0 + R, :] = y
        _stats_update(ostats_ref, b, acc1, acc2)

    return pl.pallas_call(
        body, grid=(NB,),
        in_specs=[pl.BlockSpec((2, 1, IMG, IMG), lambda b: (0, b, 0, 0)),
                  pl.BlockSpec(W3.shape, lambda b: (0, 0, 0))],
        out_specs=(pl.BlockSpec((1, Cout, Hp, Wd), lambda b: (b, 0, 0, 0)),
                   pl.BlockSpec((2, Cout), lambda b: (0, 0))),
        out_shape=(jax.ShapeDtypeStruct((NB, Cout, Hp, IMG), jnp.float32),
                   jax.ShapeDtypeStruct((2, Cout), jnp.float32)))(grid_parts, W3)


def _down(taps, bn_stack, WdT, Cout, H, cnt):
    H2, W2 = H // 2, H // 2
    Hp2 = H2 + 2
    Cin = int(taps[0].shape[1])
    R = min(R_STRIP, H2)

    def body(t0, t1, t2, t3, bn_ref, w_ref, out_ref, ostats_ref):
        b = pl.program_id(0)
        a, bb = _bn_ab(bn_ref[...], cnt)
        out_ref[0, :, 0, :] = jnp.zeros((Cout, W2), jnp.float32)
        out_ref[0, :, Hp2 - 1, :] = jnp.zeros((Cout, W2), jnp.float32)
        acc1 = jnp.zeros((Cout,), jnp.float32)
        acc2 = jnp.zeros((Cout,), jnp.float32)
        for i0 in range(0, H2, R):
            parts = []
            for t in (t0, t1, t2, t3):
                s = t[0, :, i0:i0 + R, :]
                parts.append(jnp.maximum(
                    s * a[:, None, None] + bb[:, None, None], 0.0))
            Xd = jnp.concatenate(parts, axis=0)
            y = _dot(w_ref[...], Xd.reshape(4 * Cin, R * W2)).reshape(Cout, R, W2)
            acc1 = acc1 + y.sum((1, 2))
            acc2 = acc2 + (y * y).sum((1, 2))
            out_ref[0, :, 1 + i0:1 + i0 + R, :] = y
        _stats_update(ostats_ref, b, acc1, acc2)

    tspec = pl.BlockSpec((1, Cin, H2, W2), lambda b: (b, 0, 0, 0))
    return pl.pallas_call(
        body, grid=(NB,),
        in_specs=[tspec, tspec, tspec, tspec,
                  pl.BlockSpec((4, Cin), lambda b: (0, 0)),
                  pl.BlockSpec(WdT.shape, lambda b: (0, 0))],
        out_specs=(pl.BlockSpec((1, Cout, Hp2, W2), lambda b: (b, 0, 0, 0)),
                   pl.BlockSpec((2, Cout), lambda b: (0, 0))),
        out_shape=(jax.ShapeDtypeStruct((NB, Cout, Hp2, W2), jnp.float32),
                   jax.ShapeDtypeStruct((2, Cout), jnp.float32)))(
            *taps, bn_stack, WdT)


def _up(y, bn_stack, Wu4T, Cout, h, cnt):
    w_ = h
    Cin = int(y.shape[1])
    R = min(R_STRIP, h)

    def body(y_ref, bn_ref, w_ref, out_ref, ostats_ref):
        b = pl.program_id(0)
        a, bb = _bn_ab(bn_ref[...], cnt)
        acc1 = jnp.zeros((Cout,), jnp.float32)
        acc2 = jnp.zeros((Cout,), jnp.float32)
        for i0 in range(0, h, R):
            ys = y_ref[0, :, 1 + i0:1 + i0 + R, :]
            ys = jnp.maximum(ys * a[:, None, None] + bb[:, None, None], 0.0)
            u = _dot(w_ref[...], ys.reshape(Cin, R * w_))
            u4 = u.reshape(4 * Cout, R, w_)
            s1 = u4.sum((1, 2)).reshape(4, Cout)
            s2 = (u4 * u4).sum((1, 2)).reshape(4, Cout)
            acc1 = acc1 + s1.sum(0)
            acc2 = acc2 + s2.sum(0)
            out_ref[0, :, i0:i0 + R, :] = u4
        _stats_update(ostats_ref, b, acc1, acc2)

    return pl.pallas_call(
        body, grid=(NB,),
        in_specs=[pl.BlockSpec((1, Cin, h + 2, w_), lambda b: (b, 0, 0, 0)),
                  pl.BlockSpec((4, Cin), lambda b: (0, 0)),
                  pl.BlockSpec(Wu4T.shape, lambda b: (0, 0))],
        out_specs=(pl.BlockSpec((1, 4 * Cout, h, w_), lambda b: (b, 0, 0, 0)),
                   pl.BlockSpec((2, Cout), lambda b: (0, 0))),
        out_shape=(jax.ShapeDtypeStruct((NB, 4 * Cout, h, w_), jnp.float32),
                   jax.ShapeDtypeStruct((2, Cout), jnp.float32)))(y, bn_stack, Wu4T)


def _w3(w):
    return w.transpose(0, 3, 1, 2).reshape(3, w.shape[3], 3 * w.shape[2])


def _bn_stack(stats, bnp):
    return jnp.concatenate([stats, bnp["scale"][None, :], bnp["bias"][None, :]], 0)


def _block(xs_list, p, H, cnt):
    Cout = int(p["conv2"].shape[-1])
    t, st = _fused_conv(xs_list, _w3(p["conv1"]), Cout, H, cnt)
    bn2 = _bn_stack(st, p["bn2"])
    if "nin" in p:
        nin = p["nin"][0, 0]
        off = 0
        res_nin = []
        for arr, _ in xs_list:
            c = int(arr.shape[1])
            res_nin.append((arr, nin[off:off + c].T))
            off += c
        return _fused_conv([(t, bn2)], _w3(p["conv2"]), Cout, H, cnt,
                           res_nin=res_nin)
    return _fused_conv([(t, bn2)], _w3(p["conv2"]), Cout, H, cnt,
                       res_ident=xs_list[0][0])


def _unet(x, sx, p, H):
    cnt = float(NB * H * H)
    for bp in p["blocks"]:
        x, sx = _block([(x, _bn_stack(sx, bp["bn1"]))], bp, H, cnt)
    if "inner" in p:
        Cd = int(p["down_w"].shape[-1])
        wd = p["down_w"].transpose(3, 0, 1, 2).reshape(Cd, 4 * p["down_w"].shape[2])
        taps = [x[:, :, 1 + dy:1 + dy + H:2, dx::2]
                for dy in (0, 1) for dx in (0, 1)]
        y, sy = _down(taps, _bn_stack(sx, p["bn_down"]), wd, Cd, H, cnt)
        y, sy = _unet(y, sy, p["inner"], H // 2)
        wu = p["up_w"]
        Cu = int(wu.shape[-1])
        wu4 = wu[::-1, ::-1].transpose(0, 1, 3, 2).reshape(4 * Cu, wu.shape[2])
        cnt_in = float(NB * (H // 2) * (H // 2))
        ur, su = _up(y, _bn_stack(sy, p["bn_up"]), wu4, Cu, H // 2, cnt_in)
        h2 = H // 2
        u = (ur.reshape(NB, 2, 2, Cu, h2, h2)
             .transpose(0, 3, 4, 1, 5, 2).reshape(NB, Cu, H, H))
        u = jnp.pad(u, ((0, 0), (0, 0), (1, 1), (0, 0)))
        C0 = int(x.shape[1])
        bp0 = p["post"][0]
        sa = _bn_stack(sx, {"scale": bp0["bn1"]["scale"][:C0],
                            "bias": bp0["bn1"]["bias"][:C0]})
        sb = _bn_stack(su, {"scale": bp0["bn1"]["scale"][C0:],
                            "bias": bp0["bn1"]["bias"][C0:]})
        x, sx = _block([(x, sa), (u, sb)], bp0, H, cnt)
        x, sx = _block([(x, _bn_stack(sx, p["post"][1]["bn1"]))],
                       p["post"][1], H, cnt)
    return x, sx


def _head(rows, parity, bn_stack, Wl, bl, cnt):
    C = 16
    NC = int(Wl.shape[-1])
    CH = 6400
    n_chunks = NPAD // CH

    def body(r_ref, p_ref, bn_ref, w_ref, b_ref, out_ref):
        a, bb = _bn_ab(bn_ref[...], cnt)
        r = r_ref[...]
        z = jnp.where(p_ref[...] == 1, r[:, C:2 * C], r[:, 0:C])
        z = jnp.maximum(z * a + bb, 0.0)
        out_ref[...] = _dot(z, w_ref[...]) + b_ref[...]

    return pl.pallas_call(
        body, grid=(n_chunks,),
        in_specs=[pl.BlockSpec((CH, 2 * C), lambda i: (i, 0)),
                  pl.BlockSpec((CH, 1), lambda i: (i, 0)),
                  pl.BlockSpec((4, C), lambda i: (0, 0)),
                  pl.BlockSpec((C, NC), lambda i: (0, 0)),
                  pl.BlockSpec((1, NC), lambda i: (0, 0))],
        out_specs=pl.BlockSpec((CH, NC), lambda i: (i, 0)),
        out_shape=jax.ShapeDtypeStruct((NPAD, NC), jnp.float32))(
            rows, parity, bn_stack, Wl, bl)


def _indices(coords_t):
    def body(c_ref, out_ref, par_ref):
        c0 = c_ref[0, :]
        c1 = c_ref[1, :]
        c2 = c_ref[2, :]
        cell = c0 * (IMG * IMG) + c1 * IMG + c2
        out_ref[0, :] = cell
        out_ref[1, :] = cell >> 1
        par_ref[...] = (cell & 1).reshape(NPAD, 1)

    return pl.pallas_call(
        body,
        in_specs=[pl.BlockSpec((3, NPAD), lambda: (0, 0))],
        out_specs=(pl.BlockSpec((2, NPAD), lambda: (0, 0)),
                   pl.BlockSpec((NPAD, 1), lambda: (0, 0))),
        out_shape=(jax.ShapeDtypeStruct((2, NPAD), jnp.int32),
                   jax.ShapeDtypeStruct((NPAD, 1), jnp.int32)))(coords_t)


def _scatter_grid(idx_s, feats_p):
    grid = jnp.zeros((NB * IMG * IMG,), jnp.float32)
    grid = grid.at[idx_s].add(feats_p)
    half = jnp.stack([grid, jnp.zeros_like(grid)], 0)
    return half.reshape(2, NB, IMG, IMG)


def _gather_rows(table, grow):
    return table[grow]


def kernel(coords, features, params):
    N = coords.shape[0]
    coords_p = jnp.pad(coords, ((0, NPAD - N), (0, 0)))
    feats_p = jnp.pad(features[:, 0], ((0, NPAD - N),))
    idx, parity = _indices(coords_p.T)
    grid_parts = _scatter_grid(idx[0], feats_p)

    x, sx = _stem(grid_parts, _w3(params["stem"]), 16)
    x, sx = _unet(x, sx, params["unet"], IMG)

    table = (x[:, :, 1:IMG + 1, :].transpose(0, 2, 3, 1)
             .reshape(NB * IMG * IMG // 2, 32))
    rows = _gather_rows(table, idx[1])
    out = _head(rows, parity, _bn_stack(sx, params["bn_out"]),
                params["linear_w"], params["linear_b"][None, :],
                float(NB * IMG * IMG))
    return out[:N]
